# baseline (speedup 1.0000x reference)
import functools

import jax
import jax.numpy as jnp
from jax import lax
from jax.experimental import pallas as pl
from jax.experimental.pallas import tpu as pltpu
from jax.experimental.pallas import tpu_sc as plsc

B = 4
N = 4096
C = 2048
NW = 32
CPW = 256
LG = CPW // 16
CB = C // CPW
UNITS = B * CB
UPW = UNITS // NW
RCHUNK = 32
NCHUNK = N // RCHUNK
T = UPW * NCHUNK
SLOTS = 4


def _sc_cumsum(x2):
    mesh = plsc.VectorSubcoreMesh(core_axis_name="c", subcore_axis_name="s")

    @functools.partial(
        pl.kernel,
        mesh=mesh,
        out_type=jax.ShapeDtypeStruct((B * N, C), jnp.float32),
        scratch_types=(
            [pltpu.VMEM((RCHUNK, CPW), jnp.float32)] * (2 * SLOTS)
            + [pltpu.SemaphoreType.DMA] * (2 * SLOTS)
        ),
    )
    def k(x_hbm, out_hbm, *bufs):
        ins = bufs[:SLOTS]
        outs = bufs[SLOTS:2 * SLOTS]
        sis = bufs[2 * SLOTS:3 * SLOTS]
        sos = bufs[3 * SLOTS:]
        wid = lax.axis_index("s") * 2 + lax.axis_index("c")

        def src(t):
            u = t // NCHUNK
            unit = wid * UPW + u
            b = unit // CB
            c0 = pl.multiple_of((unit % CB) * CPW, CPW)
            r0 = pl.multiple_of(b * N + (t % NCHUNK) * RCHUNK, RCHUNK)
            return (pl.ds(r0, RCHUNK), pl.ds(c0, CPW))

        for p in range(SLOTS):
            pltpu.async_copy(x_hbm.at[src(p)], ins[p], sis[p])

        def chunk(t, accs, slot):
            bi, bo = ins[slot], outs[slot]
            pltpu.make_async_copy(x_hbm.at[src(t)], bi, sis[slot]).wait()
            @pl.when(t >= SLOTS)
            def _():
                pltpu.make_async_copy(bo, out_hbm.at[src(t)], sos[slot]).wait()

            fresh = (t % NCHUNK) == 0
            accs = tuple(jnp.where(fresh, jnp.zeros((16,), jnp.float32), a)
                         for a in accs)

            def body(r, accs):
                new = []
                for g in range(LG):
                    v = bi[r, pl.ds(g * 16, 16)]
                    a = accs[g] + v
                    bo[r, pl.ds(g * 16, 16)] = a
                    new.append(a)
                return tuple(new)

            accs = lax.fori_loop(0, RCHUNK, body, accs)
            pltpu.async_copy(bo, out_hbm.at[src(t)], sos[slot])

            @pl.when(t + SLOTS < T)
            def _():
                pltpu.async_copy(x_hbm.at[src(t + SLOTS)], bi, sis[slot])

            return accs

        def group(j, accs):
            for s in range(SLOTS):
                accs = chunk(SLOTS * j + s, accs, s)
            return accs

        accs0 = tuple(jnp.zeros((16,), jnp.float32) for _ in range(LG))
        lax.fori_loop(0, T // SLOTS, group, accs0)

        for s in range(SLOTS):
            t = jnp.int32(T - SLOTS + s)
            pltpu.make_async_copy(outs[s], out_hbm.at[src(t)], sos[s]).wait()

    return k(x2)


def kernel(x):
    orig_dtype = x.dtype
    x2 = x.astype(jnp.float32).reshape(B * N, C)
    out = _sc_cumsum(x2)
    return out.reshape(B, N, C).astype(orig_dtype)


# speedup vs baseline: 1.0119x; 1.0119x over previous
import functools

import jax
import jax.numpy as jnp
from jax import lax
from jax.experimental import pallas as pl
from jax.experimental.pallas import tpu as pltpu
from jax.experimental.pallas import tpu_sc as plsc

B = 4
N = 4096
C = 2048
NW = 32
CPW = 256
LG = CPW // 16
CB = C // CPW
UNITS = B * CB
UPW = UNITS // NW
RCHUNK = 64
NCHUNK = N // RCHUNK
T = UPW * NCHUNK
SLOTS = 2


def _sc_cumsum(x2):
    mesh = plsc.VectorSubcoreMesh(core_axis_name="c", subcore_axis_name="s")

    @functools.partial(
        pl.kernel,
        mesh=mesh,
        out_type=jax.ShapeDtypeStruct((B * N, C), jnp.float32),
        scratch_types=(
            [pltpu.VMEM((RCHUNK, CPW), jnp.float32)] * (2 * SLOTS)
            + [pltpu.SemaphoreType.DMA] * (2 * SLOTS)
        ),
    )
    def k(x_hbm, out_hbm, *bufs):
        ins = bufs[:SLOTS]
        outs = bufs[SLOTS:2 * SLOTS]
        sis = bufs[2 * SLOTS:3 * SLOTS]
        sos = bufs[3 * SLOTS:]
        wid = lax.axis_index("s") * 2 + lax.axis_index("c")

        def src(t):
            u = t // NCHUNK
            unit = wid * UPW + u
            b = unit // CB
            c0 = pl.multiple_of((unit % CB) * CPW, CPW)
            r0 = pl.multiple_of(b * N + (t % NCHUNK) * RCHUNK, RCHUNK)
            return (pl.ds(r0, RCHUNK), pl.ds(c0, CPW))

        for p in range(SLOTS):
            pltpu.async_copy(x_hbm.at[src(p)], ins[p], sis[p])

        def chunk(t, accs, slot):
            bi, bo = ins[slot], outs[slot]
            pltpu.make_async_copy(x_hbm.at[src(t)], bi, sis[slot]).wait()
            @pl.when(t >= SLOTS)
            def _():
                pltpu.make_async_copy(bo, out_hbm.at[src(t)], sos[slot]).wait()

            fresh = (t % NCHUNK) == 0
            accs = tuple(jnp.where(fresh, jnp.zeros((16,), jnp.float32), a)
                         for a in accs)

            def body(r, accs):
                new = []
                for g in range(LG):
                    v = bi[r, pl.ds(g * 16, 16)]
                    a = accs[g] + v
                    bo[r, pl.ds(g * 16, 16)] = a
                    new.append(a)
                return tuple(new)

            accs = lax.fori_loop(0, RCHUNK, body, accs)
            pltpu.async_copy(bo, out_hbm.at[src(t)], sos[slot])

            @pl.when(t + SLOTS < T)
            def _():
                pltpu.async_copy(x_hbm.at[src(t + SLOTS)], bi, sis[slot])

            return accs

        def group(j, accs):
            for s in range(SLOTS):
                accs = chunk(SLOTS * j + s, accs, s)
            return accs

        accs0 = tuple(jnp.zeros((16,), jnp.float32) for _ in range(LG))
        lax.fori_loop(0, T // SLOTS, group, accs0)

        for s in range(SLOTS):
            t = jnp.int32(T - SLOTS + s)
            pltpu.make_async_copy(outs[s], out_hbm.at[src(t)], sos[s]).wait()

    return k(x2)


def kernel(x):
    orig_dtype = x.dtype
    x2 = x.astype(jnp.float32).reshape(B * N, C)
    out = _sc_cumsum(x2)
    return out.reshape(B, N, C).astype(orig_dtype)
